# baseline (speedup 1.0000x reference)
import functools

import jax
import jax.numpy as jnp
from jax import lax
from jax.experimental import pallas as pl
from jax.experimental.pallas import tpu as pltpu
from jax.experimental.pallas import tpu_sc as plsc

N_EMBEDS = 1000000
EMBED_DIM = 64
BATCH = 16384
FIELDS = 26

_NC = 2
_NS = 16
_NW = _NC * _NS
_B = BATCH * FIELDS
_BPW = _B // _NW
_NBT = BATCH // 128
_BT_PER_W = _NBT // _NW
_NGROUP = _BT_PER_W * FIELDS


def _emb_kernel(idx_hbm, table_hbm, out_hbm, idx_v, sel_v, rows_v, tile_v,
                gsem, wsem):
    wid = lax.axis_index("s") * _NC + lax.axis_index("c")
    base = wid * _BPW
    pltpu.sync_copy(idx_hbm.at[pl.ds(base, _BPW)], idx_v)
    lane = lax.iota(jnp.int32, 16)

    def stage_sel(g, sel_r):
        bt_local = g // FIELDS
        f = g - bt_local * FIELDS
        ibase = bt_local * (128 * FIELDS) + f
        for m in range(8):
            iv = ibase + (m * 16 + lane) * FIELDS
            sel_r[pl.ds(m * 16, 16)] = plsc.load_gather(idx_v, [iv])

    def fire_gather(sel_r, rows_r, sem):
        return pltpu.async_copy(table_hbm.at[sel_r], rows_r, sem)

    def shuffle(rows_r, tile_r):
        for d in range(EMBED_DIM):
            for kb in range(8):
                v = plsc.load_gather(rows_r, [kb * 16 + lane, d + lane * 0])
                tile_r[d // 8, d % 8, pl.ds(kb * 16, 16)] = v

    def fire_write(g, tile_r, sem):
        bt_local = g // FIELDS
        f = g - bt_local * FIELDS
        bt = wid * _BT_PER_W + bt_local
        return pltpu.async_copy(tile_r, out_hbm.at[f, :, bt], sem)

    def drain_write(tile_r, sem):
        pltpu.make_async_copy(tile_r, out_hbm.at[0, :, 0], sem).wait()

    def pipe_body(t, carry):
        g0 = 2 * t
        g1 = g0 + 1
        stage_sel(g0, sel_v.at[0])
        c0 = fire_gather(sel_v.at[0], rows_v.at[0], gsem.at[0])
        stage_sel(g1, sel_v.at[1])
        c1 = fire_gather(sel_v.at[1], rows_v.at[1], gsem.at[1])
        c0.wait()

        @pl.when(t > 0)
        def _():
            drain_write(tile_v.at[0], wsem.at[0])

        shuffle(rows_v.at[0], tile_v.at[0])
        fire_write(g0, tile_v.at[0], wsem.at[0])
        c1.wait()

        @pl.when(t > 0)
        def _():
            drain_write(tile_v.at[1], wsem.at[1])

        shuffle(rows_v.at[1], tile_v.at[1])
        fire_write(g1, tile_v.at[1], wsem.at[1])
        return carry

    lax.fori_loop(0, _NGROUP // 2, pipe_body, 0)
    for b in range(2):
        drain_write(tile_v.at[b], wsem.at[b])


_TROWS = 8192
_TGRID = (N_EMBEDS + _TROWS - 1) // _TROWS


def _transpose_body(i_ref, o_ref):
    t3 = i_ref[...].T.reshape(_TROWS // 2, 2, EMBED_DIM)
    o_ref[...] = jnp.concatenate([t3[:, 0, :], t3[:, 1, :]], axis=1)


def _tc_transpose(w_t):
    return pl.pallas_call(
        _transpose_body,
        grid=(_TGRID,),
        in_specs=[pl.BlockSpec((EMBED_DIM, _TROWS), lambda j: (0, j))],
        out_specs=pl.BlockSpec((_TROWS // 2, 128), lambda j: (j, 0)),
        out_shape=jax.ShapeDtypeStruct((N_EMBEDS // 2, 128), jnp.float32),
    )(w_t)


@jax.jit
def _run(idx_flat, weights):
    f = functools.partial(
        pl.kernel,
        mesh=plsc.VectorSubcoreMesh(core_axis_name="c", subcore_axis_name="s"),
        out_type=jax.ShapeDtypeStruct((FIELDS, 8, _NBT, 8, 128), jnp.float32),
        scratch_types=[
            pltpu.VMEM((_BPW,), jnp.int32),
            pltpu.VMEM((2, 128), jnp.int32),
            pltpu.VMEM((2, 128, EMBED_DIM), jnp.float32),
            pltpu.VMEM((2, 8, 8, 128), jnp.float32),
            pltpu.SemaphoreType.DMA((2,)),
            pltpu.SemaphoreType.DMA((2,)),
        ],
        compiler_params=pltpu.CompilerParams(use_tc_tiling_on_sc=False, needs_layout_passes=False),
    )(_emb_kernel)
    return f(idx_flat, weights)


def kernel(x, weights):
    table_lin = _tc_transpose(weights.T).reshape(N_EMBEDS, EMBED_DIM)
    out5 = _run(x.reshape(-1), table_lin)
    return jnp.transpose(out5, (2, 4, 0, 1, 3)).reshape(BATCH, FIELDS, EMBED_DIM)


# speedup vs baseline: 1.3365x; 1.3365x over previous
import functools

import jax
import jax.numpy as jnp
from jax import lax
from jax.experimental import pallas as pl
from jax.experimental.pallas import tpu as pltpu
from jax.experimental.pallas import tpu_sc as plsc

N_EMBEDS = 1000000
EMBED_DIM = 64
BATCH = 16384
FIELDS = 26

_NC = 2
_NS = 16
_NW = _NC * _NS
_B = BATCH * FIELDS
_BPW = _B // _NW
_GRP = 128
_CHUNK = 512
_NGRP = _CHUNK // _GRP
_NCHUNK = _BPW // _CHUNK


def _emb_kernel(idx_hbm, table_hbm, out_hbm, idx_v, rows_v, gsem, wsem):
    wid = lax.axis_index("s") * _NC + lax.axis_index("c")
    base = wid * _BPW
    pltpu.sync_copy(idx_hbm.at[pl.ds(base, _BPW)], idx_v)

    def fire_gathers(j, b):
        off = j * _CHUNK
        return [
            pltpu.async_copy(
                table_hbm.at[idx_v.at[pl.ds(off + g * _GRP, _GRP)]],
                rows_v.at[b, pl.ds(g * _GRP, _GRP)],
                gsem.at[b],
            )
            for g in range(_NGRP)
        ]

    gathers = [None, None]
    writes = [None, None]
    for j in range(_NCHUNK):
        b = j % 2
        if writes[b] is not None:
            writes[b].wait()
            writes[b] = None
        gathers[b] = fire_gathers(j, b)
        pb = 1 - b
        if gathers[pb] is not None:
            for c in gathers[pb]:
                c.wait()
            gathers[pb] = None
            writes[pb] = pltpu.async_copy(
                rows_v.at[pb],
                out_hbm.at[pl.ds(base + (j - 1) * _CHUNK, _CHUNK)],
                wsem.at[pb],
            )
    lb = (_NCHUNK - 1) % 2
    for c in gathers[lb]:
        c.wait()
    writes[lb] = pltpu.async_copy(
        rows_v.at[lb],
        out_hbm.at[pl.ds(base + (_NCHUNK - 1) * _CHUNK, _CHUNK)],
        wsem.at[lb],
    )
    for w in writes:
        if w is not None:
            w.wait()


_TROWS = 2048
_TGRID = (N_EMBEDS + _TROWS - 1) // _TROWS


def _transpose_body(i_ref, o_ref):
    t3 = i_ref[...].T.reshape(_TROWS // 2, 2, EMBED_DIM)
    o_ref[...] = jnp.concatenate([t3[:, 0, :], t3[:, 1, :]], axis=1)


def _tc_transpose(w_t):
    return pl.pallas_call(
        _transpose_body,
        grid=(_TGRID,),
        in_specs=[pl.BlockSpec((EMBED_DIM, _TROWS), lambda j: (0, j))],
        out_specs=pl.BlockSpec((_TROWS // 2, 128), lambda j: (j, 0)),
        out_shape=jax.ShapeDtypeStruct((N_EMBEDS // 2, 128), jnp.float32),
    )(w_t)


@jax.jit
def _run(idx_flat, weights):
    f = functools.partial(
        pl.kernel,
        mesh=plsc.VectorSubcoreMesh(core_axis_name="c", subcore_axis_name="s"),
        out_type=jax.ShapeDtypeStruct((_B, EMBED_DIM), jnp.float32),
        scratch_types=[
            pltpu.VMEM((_BPW,), jnp.int32),
            pltpu.VMEM((2, _CHUNK, EMBED_DIM), jnp.float32),
            pltpu.SemaphoreType.DMA((2,)),
            pltpu.SemaphoreType.DMA((2,)),
        ],
        compiler_params=pltpu.CompilerParams(use_tc_tiling_on_sc=False),
    )(_emb_kernel)
    return f(idx_flat, weights)


def kernel(x, weights):
    table_lin = _tc_transpose(weights.T).reshape(N_EMBEDS, EMBED_DIM)
    out = _run(x.reshape(-1), table_lin)
    return out.reshape(BATCH, FIELDS, EMBED_DIM)


# speedup vs baseline: 1.5828x; 1.1842x over previous
import functools

import jax
import jax.numpy as jnp
from jax import lax
from jax.experimental import pallas as pl
from jax.experimental.pallas import tpu as pltpu
from jax.experimental.pallas import tpu_sc as plsc

N_EMBEDS = 1000000
EMBED_DIM = 64
BATCH = 16384
FIELDS = 26

_NC = 2
_NS = 16
_NW = _NC * _NS
_B = BATCH * FIELDS
_BPW = _B // _NW
_GRP = 128
_CHUNK = 512
_NGRP = _CHUNK // _GRP
_NCHUNK = _BPW // _CHUNK


def _emb_kernel(idx_hbm, table_hbm, out_hbm, idx_v, rows_v, gsem, wsem):
    wid = lax.axis_index("s") * _NC + lax.axis_index("c")
    base = wid * _BPW
    pltpu.sync_copy(idx_hbm.at[pl.ds(base, _BPW)], idx_v)

    def fire_gathers(j, b):
        off = j * _CHUNK
        return [
            pltpu.async_copy(
                table_hbm.at[idx_v.at[pl.ds(off + g * _GRP, _GRP)]],
                rows_v.at[b, pl.ds(g * _GRP, _GRP)],
                gsem.at[b],
            )
            for g in range(_NGRP)
        ]

    gathers = [None, None]
    writes = [None, None]
    for j in range(_NCHUNK):
        b = j % 2
        if writes[b] is not None:
            writes[b].wait()
            writes[b] = None
        gathers[b] = fire_gathers(j, b)
        pb = 1 - b
        if gathers[pb] is not None:
            for c in gathers[pb]:
                c.wait()
            gathers[pb] = None
            writes[pb] = pltpu.async_copy(
                rows_v.at[pb],
                out_hbm.at[pl.ds(base + (j - 1) * _CHUNK, _CHUNK)],
                wsem.at[pb],
            )
    lb = (_NCHUNK - 1) % 2
    for c in gathers[lb]:
        c.wait()
    writes[lb] = pltpu.async_copy(
        rows_v.at[lb],
        out_hbm.at[pl.ds(base + (_NCHUNK - 1) * _CHUNK, _CHUNK)],
        wsem.at[lb],
    )
    for w in writes:
        if w is not None:
            w.wait()


_TROWS = 16384
_TGRID = (N_EMBEDS + _TROWS - 1) // _TROWS


def _transpose_body(i_ref, o_ref):
    t3 = i_ref[...].T.reshape(_TROWS // 2, 2, EMBED_DIM)
    o_ref[...] = jnp.concatenate([t3[:, 0, :], t3[:, 1, :]], axis=1)


def _tc_transpose(w_t):
    return pl.pallas_call(
        _transpose_body,
        grid=(_TGRID,),
        in_specs=[pl.BlockSpec((EMBED_DIM, _TROWS), lambda j: (0, j))],
        out_specs=pl.BlockSpec((_TROWS // 2, 128), lambda j: (j, 0)),
        out_shape=jax.ShapeDtypeStruct((N_EMBEDS // 2, 128), jnp.float32),
    )(w_t)


@jax.jit
def _run(idx_flat, weights):
    f = functools.partial(
        pl.kernel,
        mesh=plsc.VectorSubcoreMesh(core_axis_name="c", subcore_axis_name="s"),
        out_type=jax.ShapeDtypeStruct((_B, EMBED_DIM), jnp.float32),
        scratch_types=[
            pltpu.VMEM((_BPW,), jnp.int32),
            pltpu.VMEM((2, _CHUNK, EMBED_DIM), jnp.float32),
            pltpu.SemaphoreType.DMA((2,)),
            pltpu.SemaphoreType.DMA((2,)),
        ],
        compiler_params=pltpu.CompilerParams(use_tc_tiling_on_sc=False),
    )(_emb_kernel)
    return f(idx_flat, weights)


def kernel(x, weights):
    table_lin = _tc_transpose(weights.T).reshape(N_EMBEDS, EMBED_DIM)
    out = _run(x.reshape(-1), table_lin)
    return out.reshape(BATCH, FIELDS, EMBED_DIM)
